# Initial kernel scaffold; baseline (speedup 1.0000x reference)
import jax
import jax.numpy as jnp
from jax.experimental import pallas as pl


def kernel(input_ids, transition_table, fill_values):
    raise NotImplementedError("write your pallas kernel here")


# speedup vs baseline: 18.6440x; 18.6440x over previous
import functools

import jax
import jax.numpy as jnp
from jax import lax
from jax.experimental import pallas as pl
from jax.experimental.pallas import tpu as pltpu
from jax.experimental.pallas import tpu_sc as plsc

L = 16
NC = 2
NS = 16
NW = NC * NS
CHUNK = 2048


def _sc_build(n_tokens: int, vocab: int):
    per_w = n_tokens // NW
    n_chunks = per_w // CHUNK
    mesh = plsc.VectorSubcoreMesh(core_axis_name="c", subcore_axis_name="s")

    @functools.partial(
        pl.kernel,
        out_type=jax.ShapeDtypeStruct((n_tokens * vocab,), jnp.float32),
        mesh=mesh,
        scratch_types=[
            pltpu.VMEM((vocab,), jnp.int32),
            pltpu.VMEM((CHUNK,), jnp.int32),
            pltpu.VMEM((CHUNK * vocab,), jnp.float32),
            pltpu.VMEM((L,), jnp.float32),
            pltpu.VMEM((L,), jnp.float32),
        ],
        compiler_params=pltpu.CompilerParams(needs_layout_passes=False),
    )
    def sc_kernel(ids_hbm, table_hbm, f0_hbm, f1_hbm, out_hbm,
                  table_v, idx_v, out_v, f0_v, f1_v):
        wid = lax.axis_index("s") * NC + lax.axis_index("c")
        tok0 = wid * per_w
        pltpu.sync_copy(table_hbm, table_v)
        pltpu.sync_copy(f0_hbm, f0_v)
        pltpu.sync_copy(f1_hbm, f1_v)
        f0 = f0_v[...]
        f1 = f1_v[...]
        iota = lax.iota(jnp.int32, L)

        @pl.loop(0, CHUNK * vocab // L, unroll=8)
        def _(j):
            out_v[pl.ds(j * L, L)] = f0

        @pl.loop(0, n_chunks)
        def _(c):
            base = tok0 + c * CHUNK
            pltpu.sync_copy(ids_hbm.at[pl.ds(base, CHUNK)], idx_v)

            @pl.loop(0, CHUNK // L, unroll=4)
            def _(j):
                ids16 = idx_v[pl.ds(j * L, L)]
                next16 = plsc.load_gather(table_v, [ids16])
                flat16 = (iota + j * L) * vocab + next16
                plsc.store_scatter(out_v, [flat16], f1)

            pltpu.sync_copy(out_v, out_hbm.at[pl.ds(base * vocab, CHUNK * vocab)])

            @pl.loop(0, CHUNK // L, unroll=4)
            def _(j):
                ids16 = idx_v[pl.ds(j * L, L)]
                next16 = plsc.load_gather(table_v, [ids16])
                flat16 = (iota + j * L) * vocab + next16
                plsc.store_scatter(out_v, [flat16], f0)

    return sc_kernel


def kernel(input_ids, transition_table, fill_values):
    batch, seq = input_ids.shape
    vocab = transition_table.shape[0]
    n = batch * seq
    ids_flat = input_ids.reshape(n)
    f0 = jnp.broadcast_to(fill_values[0], (L,)).astype(jnp.float32)
    f1 = jnp.broadcast_to(fill_values[1], (L,)).astype(jnp.float32)
    out = _sc_build(n, vocab)(ids_flat, transition_table, f0, f1)
    return out.reshape(batch, seq, vocab)
